# baseline (speedup 1.0000x reference)
import jax
import jax.numpy as jnp
from jax.experimental import pallas as pl
from jax.experimental.pallas import tpu as pltpu

_S_BLK = 1024
_NBUF = 4


def _blocks(B, S):
    edge = [128, 128, 256, 512]
    rows_total = B * S
    sizes = edge + [_S_BLK] * ((rows_total - 2 * sum(edge)) // _S_BLK) \
        + edge[::-1]
    assert sum(sizes) == rows_total
    blocks = []
    off = 0
    for r in sizes:
        blocks.append((off // S, off % S, r))
        off += r
    return blocks


def _film_pipelined(ts_ref, x_hbm, tab_hbm, o_hbm, emb_buf, in_bufs, out_bufs,
                    emb_sem, in_sems, out_sems):
    B, S, D = x_hbm.shape
    blocks = _blocks(B, S)
    M = len(blocks)

    def x_view(j):
        b, off, r = blocks[j]
        return x_hbm.at[b, pl.ds(off, r), :]

    def o_view(j):
        b, off, r = blocks[j]
        return o_hbm.at[b, pl.ds(off, r), :]

    def in_slice(j):
        return in_bufs.at[j % _NBUF, pl.ds(0, blocks[j][2])]

    def out_slice(j):
        return out_bufs.at[j % _NBUF, pl.ds(0, blocks[j][2])]

    for b in range(B):
        pltpu.make_async_copy(tab_hbm.at[ts_ref[b]], emb_buf.at[b],
                              emb_sem).start()
    for k in range(_NBUF - 1):
        pltpu.make_async_copy(x_view(k), in_slice(k), in_sems.at[k]).start()
    for b in range(B):
        pltpu.make_async_copy(tab_hbm.at[ts_ref[b]], emb_buf.at[b],
                              emb_sem).wait()

    for j in range(M):
        slot = j % _NBUF
        nxt = j + _NBUF - 1
        if nxt < M:
            pltpu.make_async_copy(x_view(nxt), in_slice(nxt),
                                  in_sems.at[nxt % _NBUF]).start()
        pltpu.make_async_copy(x_view(j), in_slice(j), in_sems.at[slot]).wait()
        if j >= _NBUF:
            pltpu.make_async_copy(out_slice(j - _NBUF), o_view(j - _NBUF),
                                  out_sems.at[slot]).wait()
        b, _, r = blocks[j]
        shift = emb_buf[b, 0, :]
        scale = emb_buf[b, 1, :]
        out_bufs[slot, pl.ds(0, r)] = jax.nn.gelu(
            in_bufs[slot, pl.ds(0, r)] * scale + shift)
        pltpu.make_async_copy(out_slice(j), o_view(j), out_sems.at[slot]).start()

    for j in range(max(0, M - _NBUF), M):
        pltpu.make_async_copy(out_slice(j), o_view(j),
                              out_sems.at[j % _NBUF]).wait()


def kernel(x, timestep, film_table):
    B, S, D = x.shape
    table3 = film_table.reshape(film_table.shape[0], 2, D)
    out = pl.pallas_call(
        _film_pipelined,
        in_specs=[
            pl.BlockSpec(memory_space=pltpu.MemorySpace.SMEM),
            pl.BlockSpec(memory_space=pl.MemorySpace.ANY),
            pl.BlockSpec(memory_space=pl.MemorySpace.ANY),
        ],
        out_specs=pl.BlockSpec(memory_space=pl.MemorySpace.ANY),
        out_shape=jax.ShapeDtypeStruct((B, S, D), x.dtype),
        scratch_shapes=[
            pltpu.VMEM((B, 2, D), jnp.float32),
            pltpu.VMEM((_NBUF, _S_BLK, D), jnp.float32),
            pltpu.VMEM((_NBUF, _S_BLK, D), jnp.float32),
            pltpu.SemaphoreType.DMA,
            pltpu.SemaphoreType.DMA((_NBUF,)),
            pltpu.SemaphoreType.DMA((_NBUF,)),
        ],
    )(timestep, x, table3)
    return out
